# baseline (speedup 1.0000x reference)
import functools

import jax
import jax.numpy as jnp
from jax import lax
from jax.experimental import pallas as pl
from jax.experimental.pallas import tpu as pltpu
from jax.experimental.pallas import tpu_sc as plsc

N_NODES = 10000
D_FEAT = 128
N_EDGES = 320000
B = 1024

NC = 2
NS = 16
L = 16
NW = NC * NS
EPW = N_EDGES // NW
BPW = B // NW


RBLK = 2048
NBLK = 5


def _fmat_body(x_ref, w_ref, as_ref, ad_ref, wc_ref, f_ref, m_ref):
    @pl.when(pl.program_id(0) == 0)
    def _():
        ms = jnp.concatenate(
            [as_ref[...][:, None], ad_ref[...][:, None], wc_ref[...]], axis=1)
        m_ref[...] = jnp.dot(w_ref[...], ms, preferred_element_type=jnp.float32,
                             precision=lax.Precision.HIGHEST)

    f = jnp.dot(x_ref[...], m_ref[...], preferred_element_type=jnp.float32,
                precision=lax.Precision.HIGHEST)
    ft = f.T
    for c in range(4):
        f_ref[pl.ds(c * RBLK, RBLK)] = ft[c]


@jax.jit
def _fmat(x, w, a_src, a_dst, wc):
    return pl.pallas_call(
        _fmat_body,
        grid=(NBLK,),
        in_specs=[
            pl.BlockSpec((RBLK, D_FEAT), lambda i: (i, 0)),
            pl.BlockSpec((D_FEAT, 256), lambda i: (0, 0)),
            pl.BlockSpec((256,), lambda i: (0,)),
            pl.BlockSpec((256,), lambda i: (0,)),
            pl.BlockSpec((256, 2), lambda i: (0, 0)),
        ],
        out_specs=pl.BlockSpec((4 * RBLK,), lambda i: (i,)),
        out_shape=jax.ShapeDtypeStruct((NBLK * 4 * RBLK,), jnp.float32),
        scratch_shapes=[pltpu.VMEM((D_FEAT, 4), jnp.float32)],
    )(x, w, a_src, a_dst, wc)


ALEN = 10112


def _edge_body(f_hbm, ei_hbm, acc_hbm, f_v, ei_v, den_v, n0_v, n1_v):
    wid = lax.axis_index("s") * NC + lax.axis_index("c")
    base = wid * EPW
    a0 = (base // 128) * 128
    off = base - a0

    pltpu.sync_copy(f_hbm, f_v)
    pltpu.sync_copy(ei_hbm.at[:, pl.ds(a0, ALEN)], ei_v)

    zero = jnp.zeros((L,), jnp.float32)

    def zbody(i):
        den_v[pl.ds(i * L, L)] = zero
        n0_v[pl.ds(i * L, L)] = zero
        n1_v[pl.ds(i * L, L)] = zero

    plsc.parallel_loop(0, N_NODES // L, 1, unroll=4)(zbody)

    def ebody(i):
        s = ei_v[0, pl.ds(off + i * L, L)]
        d = ei_v[1, pl.ds(off + i * L, L)]
        sh = s >> 11
        sp = s + ((sh << 12) + (sh << 11))
        dh = d >> 11
        dp = d + ((dh << 12) + (dh << 11))
        av = plsc.load_gather(f_v, [sp])
        bv = plsc.load_gather(f_v, [dp + RBLK])
        h0 = plsc.load_gather(f_v, [sp + 2 * RBLK])
        h1 = plsc.load_gather(f_v, [sp + 3 * RBLK])
        z = av + bv
        e = jnp.where(z >= 0.0, z, z * jnp.float32(0.2))
        ex = jnp.exp(e)
        plsc.addupdate_scatter(den_v, [d], ex)
        plsc.addupdate_scatter(n0_v, [d], ex * h0)
        plsc.addupdate_scatter(n1_v, [d], ex * h1)

    plsc.parallel_loop(0, EPW // L, 1, unroll=8)(ebody)

    row = wid * 3 * N_NODES
    pltpu.sync_copy(den_v, acc_hbm.at[pl.ds(row, N_NODES)])
    pltpu.sync_copy(n0_v, acc_hbm.at[pl.ds(row + N_NODES, N_NODES)])
    pltpu.sync_copy(n1_v, acc_hbm.at[pl.ds(row + 2 * N_NODES, N_NODES)])


@jax.jit
def _edge_pass(f_flat, edge_index):
    mesh = plsc.VectorSubcoreMesh(core_axis_name="c", subcore_axis_name="s",
                                  num_cores=NC, num_subcores=NS)
    return pl.kernel(
        _edge_body,
        out_type=jax.ShapeDtypeStruct((NW * 3 * N_NODES,), jnp.float32),
        mesh=mesh,
        compiler_params=pltpu.CompilerParams(needs_layout_passes=False),
        scratch_types=[
            pltpu.VMEM((NBLK * 4 * RBLK,), jnp.float32),
            pltpu.VMEM((2, ALEN), jnp.int32),
            pltpu.VMEM((N_NODES,), jnp.float32),
            pltpu.VMEM((N_NODES,), jnp.float32),
            pltpu.VMEM((N_NODES,), jnp.float32),
        ],
    )(f_flat, edge_index)


_GCHUNK = 128


def _select_body(acc_hbm, idx_hbm, bc_hbm, out_hbm, idx_v, gidx_v, gath_v,
                 out_v, bc_v, sem):
    wid = lax.axis_index("s") * NC + lax.axis_index("c")
    pltpu.sync_copy(idx_hbm.at[pl.ds(wid * BPW, BPW)], idx_v)
    pltpu.sync_copy(bc_hbm, bc_v)

    i0 = idx_v[pl.ds(0, L)]
    i1 = idx_v[pl.ds(L, L)]

    def build(j):
        off = j * N_NODES
        gidx_v[pl.ds(j * 2 * L, L)] = i0 + off
        gidx_v[pl.ds(j * 2 * L + L, L)] = i1 + off

    plsc.parallel_loop(0, NW * 3, 1, unroll=4)(build)

    n_chunks = (NW * 3 * 2 * L) // _GCHUNK
    copies = [
        pltpu.async_copy(acc_hbm.at[gidx_v.at[pl.ds(j * _GCHUNK, _GCHUNK)]],
                         gath_v.at[pl.ds(j * _GCHUNK, _GCHUNK)], sem)
        for j in range(n_chunks)
    ]
    for c in copies:
        c.wait()

    zero = jnp.zeros((L,), jnp.float32)

    def red(k, carry):
        d0, d1, a0, a1, b0, b1 = carry
        base = k * 96
        d0 = d0 + gath_v[pl.ds(base, L)]
        d1 = d1 + gath_v[pl.ds(base + 16, L)]
        a0 = a0 + gath_v[pl.ds(base + 32, L)]
        a1 = a1 + gath_v[pl.ds(base + 48, L)]
        b0 = b0 + gath_v[pl.ds(base + 64, L)]
        b1 = b1 + gath_v[pl.ds(base + 80, L)]
        return d0, d1, a0, a1, b0, b1

    d0, d1, a0, a1, b0, b1 = lax.fori_loop(
        0, NW, red, (zero, zero, zero, zero, zero, zero))

    eps = jnp.float32(1e-16)
    bc0 = plsc.load_gather(bc_v, [jnp.zeros((L,), jnp.int32)])
    bc1 = plsc.load_gather(bc_v, [jnp.ones((L,), jnp.int32)])
    o00 = a0 / (d0 + eps) + bc0
    o01 = b0 / (d0 + eps) + bc1
    o10 = a1 / (d1 + eps) + bc0
    o11 = b1 / (d1 + eps) + bc1

    ii = lax.iota(jnp.int32, L) * 2
    plsc.store_scatter(out_v, [ii], o00)
    plsc.store_scatter(out_v, [ii + 1], o01)
    plsc.store_scatter(out_v, [ii + 2 * L], o10)
    plsc.store_scatter(out_v, [ii + 2 * L + 1], o11)

    pltpu.sync_copy(out_v, out_hbm.at[pl.ds(wid * 2 * BPW, 2 * BPW)])


@jax.jit
def _select_pass(acc_flat, idx, bc):
    mesh = plsc.VectorSubcoreMesh(core_axis_name="c", subcore_axis_name="s",
                                  num_cores=NC, num_subcores=NS)
    return pl.kernel(
        _select_body,
        out_type=jax.ShapeDtypeStruct((2 * B,), jnp.float32),
        mesh=mesh,
        compiler_params=pltpu.CompilerParams(needs_layout_passes=False),
        scratch_types=[
            pltpu.VMEM((BPW,), jnp.int32),
            pltpu.VMEM((NW * 3 * 2 * L,), jnp.int32),
            pltpu.VMEM((NW * 3 * 2 * L,), jnp.float32),
            pltpu.VMEM((2 * BPW,), jnp.float32),
            pltpu.VMEM((2,), jnp.float32),
            pltpu.SemaphoreType.DMA,
        ],
    )(acc_flat, idx, bc)


def kernel(img1, img2, index, x, edge_index, W, a_src, a_dst, Wc, bc):
    f_flat = _fmat(x, W, a_src, a_dst, Wc)
    acc = _edge_pass(f_flat, edge_index)
    idx = index.reshape(-1).astype(jnp.int32)
    out_flat = _select_pass(acc, idx, bc)
    return out_flat.reshape(B, 2)


# speedup vs baseline: 1.0271x; 1.0271x over previous
import functools

import jax
import jax.numpy as jnp
from jax import lax
from jax.experimental import pallas as pl
from jax.experimental.pallas import tpu as pltpu
from jax.experimental.pallas import tpu_sc as plsc

N_NODES = 10000
D_FEAT = 128
N_EDGES = 320000
B = 1024

NC = 2
NS = 16
L = 16
NW = NC * NS
EPW = N_EDGES // NW
BPW = B // NW


NPAD = 10240


def _fmat_body(x_ref, w_ref, as_ref, ad_ref, wc_ref, f_ref):
    ms = jnp.concatenate(
        [as_ref[...][:, None], ad_ref[...][:, None], wc_ref[...]], axis=1)
    m = jnp.dot(w_ref[...], ms, preferred_element_type=jnp.float32,
                precision=lax.Precision.HIGHEST)
    f = jnp.dot(x_ref[...], m, preferred_element_type=jnp.float32,
                precision=lax.Precision.HIGHEST)
    ft = f.T
    for c in range(4):
        f_ref[pl.ds(c * NPAD, N_NODES)] = ft[c]


@jax.jit
def _fmat(x, w, a_src, a_dst, wc):
    return pl.pallas_call(
        _fmat_body,
        out_shape=jax.ShapeDtypeStruct((4 * NPAD,), jnp.float32),
    )(x, w, a_src, a_dst, wc)


ALEN = 10112


def _edge_body(f_hbm, ei_hbm, acc_hbm, f_v, ei_v, den_v, n0_v, n1_v):
    wid = lax.axis_index("s") * NC + lax.axis_index("c")
    base = wid * EPW
    a0 = (base // 128) * 128
    off = base - a0

    pltpu.sync_copy(f_hbm, f_v)
    pltpu.sync_copy(ei_hbm.at[:, pl.ds(a0, ALEN)], ei_v)

    zero = jnp.zeros((L,), jnp.float32)

    def zbody(i):
        den_v[pl.ds(i * L, L)] = zero
        n0_v[pl.ds(i * L, L)] = zero
        n1_v[pl.ds(i * L, L)] = zero

    plsc.parallel_loop(0, N_NODES // L, 1, unroll=4)(zbody)

    def ebody(i):
        s = ei_v[0, pl.ds(off + i * L, L)]
        d = ei_v[1, pl.ds(off + i * L, L)]
        av = plsc.load_gather(f_v, [s])
        bv = plsc.load_gather(f_v, [d + NPAD])
        h0 = plsc.load_gather(f_v, [s + 2 * NPAD])
        h1 = plsc.load_gather(f_v, [s + 3 * NPAD])
        z = av + bv
        e = jnp.where(z >= 0.0, z, z * jnp.float32(0.2))
        ex = jnp.exp(e)
        plsc.addupdate_scatter(den_v, [d], ex)
        plsc.addupdate_scatter(n0_v, [d], ex * h0)
        plsc.addupdate_scatter(n1_v, [d], ex * h1)

    plsc.parallel_loop(0, EPW // L, 1, unroll=16)(ebody)

    row = wid * 3 * N_NODES
    pltpu.sync_copy(den_v, acc_hbm.at[pl.ds(row, N_NODES)])
    pltpu.sync_copy(n0_v, acc_hbm.at[pl.ds(row + N_NODES, N_NODES)])
    pltpu.sync_copy(n1_v, acc_hbm.at[pl.ds(row + 2 * N_NODES, N_NODES)])


@jax.jit
def _edge_pass(f_flat, edge_index):
    mesh = plsc.VectorSubcoreMesh(core_axis_name="c", subcore_axis_name="s",
                                  num_cores=NC, num_subcores=NS)
    return pl.kernel(
        _edge_body,
        out_type=jax.ShapeDtypeStruct((NW * 3 * N_NODES,), jnp.float32),
        mesh=mesh,
        compiler_params=pltpu.CompilerParams(needs_layout_passes=False),
        scratch_types=[
            pltpu.VMEM((4 * NPAD,), jnp.float32),
            pltpu.VMEM((2, ALEN), jnp.int32),
            pltpu.VMEM((N_NODES,), jnp.float32),
            pltpu.VMEM((N_NODES,), jnp.float32),
            pltpu.VMEM((N_NODES,), jnp.float32),
        ],
    )(f_flat, edge_index)


_GCHUNK = 128


def _select_body(acc_hbm, idx_hbm, bc_hbm, out_hbm, idx_v, gidx_v, gath_v,
                 out_v, bc_v, sem):
    wid = lax.axis_index("s") * NC + lax.axis_index("c")
    pltpu.sync_copy(idx_hbm.at[pl.ds(wid * BPW, BPW)], idx_v)
    pltpu.sync_copy(bc_hbm, bc_v)

    i0 = idx_v[pl.ds(0, L)]
    i1 = idx_v[pl.ds(L, L)]

    def build(j):
        off = j * N_NODES
        gidx_v[pl.ds(j * 2 * L, L)] = i0 + off
        gidx_v[pl.ds(j * 2 * L + L, L)] = i1 + off

    plsc.parallel_loop(0, NW * 3, 1, unroll=4)(build)

    n_chunks = (NW * 3 * 2 * L) // _GCHUNK
    copies = [
        pltpu.async_copy(acc_hbm.at[gidx_v.at[pl.ds(j * _GCHUNK, _GCHUNK)]],
                         gath_v.at[pl.ds(j * _GCHUNK, _GCHUNK)], sem)
        for j in range(n_chunks)
    ]
    for c in copies:
        c.wait()

    zero = jnp.zeros((L,), jnp.float32)

    def red(k, carry):
        d0, d1, a0, a1, b0, b1 = carry
        base = k * 96
        d0 = d0 + gath_v[pl.ds(base, L)]
        d1 = d1 + gath_v[pl.ds(base + 16, L)]
        a0 = a0 + gath_v[pl.ds(base + 32, L)]
        a1 = a1 + gath_v[pl.ds(base + 48, L)]
        b0 = b0 + gath_v[pl.ds(base + 64, L)]
        b1 = b1 + gath_v[pl.ds(base + 80, L)]
        return d0, d1, a0, a1, b0, b1

    d0, d1, a0, a1, b0, b1 = lax.fori_loop(
        0, NW, red, (zero, zero, zero, zero, zero, zero))

    eps = jnp.float32(1e-16)
    bc0 = plsc.load_gather(bc_v, [jnp.zeros((L,), jnp.int32)])
    bc1 = plsc.load_gather(bc_v, [jnp.ones((L,), jnp.int32)])
    o00 = a0 / (d0 + eps) + bc0
    o01 = b0 / (d0 + eps) + bc1
    o10 = a1 / (d1 + eps) + bc0
    o11 = b1 / (d1 + eps) + bc1

    ii = lax.iota(jnp.int32, L) * 2
    plsc.store_scatter(out_v, [ii], o00)
    plsc.store_scatter(out_v, [ii + 1], o01)
    plsc.store_scatter(out_v, [ii + 2 * L], o10)
    plsc.store_scatter(out_v, [ii + 2 * L + 1], o11)

    pltpu.sync_copy(out_v, out_hbm.at[pl.ds(wid * 2 * BPW, 2 * BPW)])


@jax.jit
def _select_pass(acc_flat, idx, bc):
    mesh = plsc.VectorSubcoreMesh(core_axis_name="c", subcore_axis_name="s",
                                  num_cores=NC, num_subcores=NS)
    return pl.kernel(
        _select_body,
        out_type=jax.ShapeDtypeStruct((2 * B,), jnp.float32),
        mesh=mesh,
        compiler_params=pltpu.CompilerParams(needs_layout_passes=False),
        scratch_types=[
            pltpu.VMEM((BPW,), jnp.int32),
            pltpu.VMEM((NW * 3 * 2 * L,), jnp.int32),
            pltpu.VMEM((NW * 3 * 2 * L,), jnp.float32),
            pltpu.VMEM((2 * BPW,), jnp.float32),
            pltpu.VMEM((2,), jnp.float32),
            pltpu.SemaphoreType.DMA,
        ],
    )(acc_flat, idx, bc)


def kernel(img1, img2, index, x, edge_index, W, a_src, a_dst, Wc, bc):
    f_flat = _fmat(x, W, a_src, a_dst, Wc)
    acc = _edge_pass(f_flat, edge_index)
    idx = index.reshape(-1).astype(jnp.int32)
    out_flat = _select_pass(acc, idx, bc)
    return out_flat.reshape(B, 2)


# speedup vs baseline: 1.0567x; 1.0289x over previous
import functools

import jax
import jax.numpy as jnp
from jax import lax
from jax.experimental import pallas as pl
from jax.experimental.pallas import tpu as pltpu
from jax.experimental.pallas import tpu_sc as plsc

N_NODES = 10000
D_FEAT = 128
N_EDGES = 320000
B = 1024

NC = 2
NS = 16
L = 16
NW = NC * NS
EPW = N_EDGES // NW
BPW = B // NW


NPAD = 10240


def _fmat_body(x_ref, w_ref, as_ref, ad_ref, wcf_ref, f_ref):
    wcf = wcf_ref[...]
    ms = jnp.concatenate(
        [as_ref[...][:, None], ad_ref[...][:, None],
         wcf[0:256][:, None], wcf[256:512][:, None]], axis=1)
    m = jnp.dot(w_ref[...], ms, preferred_element_type=jnp.float32,
                precision=lax.Precision.HIGHEST)
    f = jnp.dot(x_ref[...], m, preferred_element_type=jnp.float32,
                precision=lax.Precision.HIGHEST)
    ft = f.T
    for c in range(4):
        f_ref[pl.ds(c * NPAD, N_NODES)] = ft[c]


@jax.jit
def _fmat(x, w, a_src, a_dst, wc_flat):
    return pl.pallas_call(
        _fmat_body,
        out_shape=jax.ShapeDtypeStruct((4 * NPAD,), jnp.float32),
    )(x, w, a_src, a_dst, wc_flat)


ALEN = 10112


def _edge_body(f_hbm, ei_hbm, acc_hbm, f_v, ei_v, den_v, n0_v, n1_v):
    wid = lax.axis_index("s") * NC + lax.axis_index("c")
    base = wid * EPW
    a0 = (base // 128) * 128
    off = base - a0

    pltpu.sync_copy(f_hbm, f_v)
    pltpu.sync_copy(ei_hbm.at[:, pl.ds(a0, ALEN)], ei_v)

    zero = jnp.zeros((L,), jnp.float32)

    def zbody(i):
        den_v[pl.ds(i * L, L)] = zero
        n0_v[pl.ds(i * L, L)] = zero
        n1_v[pl.ds(i * L, L)] = zero

    plsc.parallel_loop(0, N_NODES // L, 1, unroll=4)(zbody)

    def ebody(i):
        s = ei_v[0, pl.ds(off + i * L, L)]
        d = ei_v[1, pl.ds(off + i * L, L)]
        av = plsc.load_gather(f_v, [s])
        bv = plsc.load_gather(f_v, [d + NPAD])
        h0 = plsc.load_gather(f_v, [s + 2 * NPAD])
        h1 = plsc.load_gather(f_v, [s + 3 * NPAD])
        z = av + bv
        e = jnp.where(z >= 0.0, z, z * jnp.float32(0.2))
        ex = jnp.exp(e)
        plsc.addupdate_scatter(den_v, [d], ex)
        plsc.addupdate_scatter(n0_v, [d], ex * h0)
        plsc.addupdate_scatter(n1_v, [d], ex * h1)

    plsc.parallel_loop(0, EPW // L, 1, unroll=8)(ebody)

    row = wid * 3 * N_NODES
    pltpu.sync_copy(den_v, acc_hbm.at[pl.ds(row, N_NODES)])
    pltpu.sync_copy(n0_v, acc_hbm.at[pl.ds(row + N_NODES, N_NODES)])
    pltpu.sync_copy(n1_v, acc_hbm.at[pl.ds(row + 2 * N_NODES, N_NODES)])


@jax.jit
def _edge_pass(f_flat, edge_index):
    mesh = plsc.VectorSubcoreMesh(core_axis_name="c", subcore_axis_name="s",
                                  num_cores=NC, num_subcores=NS)
    return pl.kernel(
        _edge_body,
        out_type=jax.ShapeDtypeStruct((NW * 3 * N_NODES,), jnp.float32),
        mesh=mesh,
        compiler_params=pltpu.CompilerParams(needs_layout_passes=False),
        scratch_types=[
            pltpu.VMEM((4 * NPAD,), jnp.float32),
            pltpu.VMEM((2, ALEN), jnp.int32),
            pltpu.VMEM((N_NODES,), jnp.float32),
            pltpu.VMEM((N_NODES,), jnp.float32),
            pltpu.VMEM((N_NODES,), jnp.float32),
        ],
    )(f_flat, edge_index)


_GCHUNK = 128


def _select_body(acc_hbm, idx_hbm, bc_hbm, out_hbm, idx_v, gidx_v, gath_v,
                 out_v, bc_v, sem):
    wid = lax.axis_index("s") * NC + lax.axis_index("c")
    pltpu.sync_copy(idx_hbm.at[pl.ds(wid * BPW, BPW)], idx_v)
    pltpu.sync_copy(bc_hbm, bc_v)

    i0 = idx_v[pl.ds(0, L)]
    i1 = idx_v[pl.ds(L, L)]

    def build(j):
        off = j * N_NODES
        gidx_v[pl.ds(j * 2 * L, L)] = i0 + off
        gidx_v[pl.ds(j * 2 * L + L, L)] = i1 + off

    plsc.parallel_loop(0, NW * 3, 1, unroll=4)(build)

    n_chunks = (NW * 3 * 2 * L) // _GCHUNK
    copies = [
        pltpu.async_copy(acc_hbm.at[gidx_v.at[pl.ds(j * _GCHUNK, _GCHUNK)]],
                         gath_v.at[pl.ds(j * _GCHUNK, _GCHUNK)], sem)
        for j in range(n_chunks)
    ]
    for c in copies:
        c.wait()

    zero = jnp.zeros((L,), jnp.float32)

    def red(k, carry):
        d0, d1, a0, a1, b0, b1 = carry
        base = k * 96
        d0 = d0 + gath_v[pl.ds(base, L)]
        d1 = d1 + gath_v[pl.ds(base + 16, L)]
        a0 = a0 + gath_v[pl.ds(base + 32, L)]
        a1 = a1 + gath_v[pl.ds(base + 48, L)]
        b0 = b0 + gath_v[pl.ds(base + 64, L)]
        b1 = b1 + gath_v[pl.ds(base + 80, L)]
        return d0, d1, a0, a1, b0, b1

    d0, d1, a0, a1, b0, b1 = lax.fori_loop(
        0, NW, red, (zero, zero, zero, zero, zero, zero))

    eps = jnp.float32(1e-16)
    bc0 = plsc.load_gather(bc_v, [jnp.zeros((L,), jnp.int32)])
    bc1 = plsc.load_gather(bc_v, [jnp.ones((L,), jnp.int32)])
    o00 = a0 / (d0 + eps) + bc0
    o01 = b0 / (d0 + eps) + bc1
    o10 = a1 / (d1 + eps) + bc0
    o11 = b1 / (d1 + eps) + bc1

    out_v[pl.ds(0, L)] = o00
    out_v[pl.ds(L, L)] = o10
    out_v[pl.ds(2 * L, L)] = o01
    out_v[pl.ds(3 * L, L)] = o11
    pltpu.sync_copy(out_v.at[pl.ds(0, BPW)], out_hbm.at[pl.ds(wid * BPW, BPW)])
    pltpu.sync_copy(out_v.at[pl.ds(BPW, BPW)],
                    out_hbm.at[pl.ds(B + wid * BPW, BPW)])


@jax.jit
def _select_pass(acc_flat, idx, bc):
    mesh = plsc.VectorSubcoreMesh(core_axis_name="c", subcore_axis_name="s",
                                  num_cores=NC, num_subcores=NS)
    return pl.kernel(
        _select_body,
        out_type=jax.ShapeDtypeStruct((2 * B,), jnp.float32),
        mesh=mesh,
        compiler_params=pltpu.CompilerParams(needs_layout_passes=False),
        scratch_types=[
            pltpu.VMEM((BPW,), jnp.int32),
            pltpu.VMEM((NW * 3 * 2 * L,), jnp.int32),
            pltpu.VMEM((NW * 3 * 2 * L,), jnp.float32),
            pltpu.VMEM((2 * BPW,), jnp.float32),
            pltpu.VMEM((2,), jnp.float32),
            pltpu.SemaphoreType.DMA,
        ],
    )(acc_flat, idx, bc)


def kernel(img1, img2, index, x, edge_index, W, a_src, a_dst, Wc, bc):
    f_flat = _fmat(x, W, a_src, a_dst, Wc.T.reshape(-1))
    acc = _edge_pass(f_flat, edge_index)
    idx = index.reshape(-1).astype(jnp.int32)
    out_flat = _select_pass(acc, idx, bc)
    return out_flat.reshape(2, B).T


# speedup vs baseline: 1.0913x; 1.0328x over previous
import functools

import jax
import jax.numpy as jnp
from jax import lax
from jax.experimental import pallas as pl
from jax.experimental.pallas import tpu as pltpu
from jax.experimental.pallas import tpu_sc as plsc

N_NODES = 10000
D_FEAT = 128
N_EDGES = 320000
B = 1024

NC = 2
NS = 16
L = 16
NW = NC * NS
EPW = N_EDGES // NW
BPW = B // NW


NPAD = 10240


def _fmat_body(x_ref, w_ref, as_ref, ad_ref, wcf_ref, f_ref):
    wcf = wcf_ref[...]
    ms = jnp.concatenate(
        [as_ref[...][:, None], ad_ref[...][:, None],
         wcf[0:256][:, None], wcf[256:512][:, None]], axis=1)
    m = jnp.dot(w_ref[...], ms, preferred_element_type=jnp.float32,
                precision=lax.Precision.HIGHEST)
    f = jnp.dot(x_ref[...], m, preferred_element_type=jnp.float32,
                precision=lax.Precision.HIGHEST)
    ft = f.T
    for c in range(4):
        f_ref[pl.ds(c * NPAD, N_NODES)] = ft[c]


@jax.jit
def _fmat(x, w, a_src, a_dst, wc_flat):
    return pl.pallas_call(
        _fmat_body,
        out_shape=jax.ShapeDtypeStruct((4 * NPAD,), jnp.float32),
    )(x, w, a_src, a_dst, wc_flat)


ALEN = 10112


def _edge_body(f_hbm, ei_hbm, acc_hbm, f_v, ei_v, den_v, n0_v, n1_v, sem):
    wid = lax.axis_index("s") * NC + lax.axis_index("c")
    base = wid * EPW
    a0 = (base // 128) * 128
    off = base - a0

    cf = pltpu.async_copy(f_hbm, f_v, sem)
    ce = pltpu.async_copy(ei_hbm.at[:, pl.ds(a0, ALEN)], ei_v, sem)

    zero = jnp.zeros((L,), jnp.float32)

    def zbody(i):
        den_v[pl.ds(i * L, L)] = zero
        n0_v[pl.ds(i * L, L)] = zero
        n1_v[pl.ds(i * L, L)] = zero

    plsc.parallel_loop(0, N_NODES // L, 1, unroll=4)(zbody)
    cf.wait()
    ce.wait()

    def ebody(i):
        s = ei_v[0, pl.ds(off + i * L, L)]
        d = ei_v[1, pl.ds(off + i * L, L)]
        av = plsc.load_gather(f_v, [s])
        bv = plsc.load_gather(f_v, [d + NPAD])
        h0 = plsc.load_gather(f_v, [s + 2 * NPAD])
        h1 = plsc.load_gather(f_v, [s + 3 * NPAD])
        z = av + bv
        e = jnp.where(z >= 0.0, z, z * jnp.float32(0.2))
        ex = jnp.exp(e)
        plsc.addupdate_scatter(den_v, [d], ex)
        plsc.addupdate_scatter(n0_v, [d], ex * h0)
        plsc.addupdate_scatter(n1_v, [d], ex * h1)

    plsc.parallel_loop(0, EPW // L, 1, unroll=8)(ebody)

    row = wid * 3 * N_NODES
    pltpu.sync_copy(den_v, acc_hbm.at[pl.ds(row, N_NODES)])
    pltpu.sync_copy(n0_v, acc_hbm.at[pl.ds(row + N_NODES, N_NODES)])
    pltpu.sync_copy(n1_v, acc_hbm.at[pl.ds(row + 2 * N_NODES, N_NODES)])


@jax.jit
def _edge_pass(f_flat, edge_index):
    mesh = plsc.VectorSubcoreMesh(core_axis_name="c", subcore_axis_name="s",
                                  num_cores=NC, num_subcores=NS)
    return pl.kernel(
        _edge_body,
        out_type=jax.ShapeDtypeStruct((NW * 3 * N_NODES,), jnp.float32),
        mesh=mesh,
        compiler_params=pltpu.CompilerParams(needs_layout_passes=False),
        scratch_types=[
            pltpu.VMEM((4 * NPAD,), jnp.float32),
            pltpu.VMEM((2, ALEN), jnp.int32),
            pltpu.VMEM((N_NODES,), jnp.float32),
            pltpu.VMEM((N_NODES,), jnp.float32),
            pltpu.VMEM((N_NODES,), jnp.float32),
            pltpu.SemaphoreType.DMA,
        ],
    )(f_flat, edge_index)


_GCHUNK = 128


def _select_body(acc_hbm, idx_hbm, bc_hbm, out_hbm, idx_v, gidx_v, gath_v,
                 out_v, bc_v, sem):
    wid = lax.axis_index("s") * NC + lax.axis_index("c")
    pltpu.sync_copy(idx_hbm.at[pl.ds(wid * BPW, BPW)], idx_v)
    pltpu.sync_copy(bc_hbm, bc_v)

    i0 = idx_v[pl.ds(0, L)]
    i1 = idx_v[pl.ds(L, L)]

    def build(j):
        off = j * N_NODES
        gidx_v[pl.ds(j * 2 * L, L)] = i0 + off
        gidx_v[pl.ds(j * 2 * L + L, L)] = i1 + off

    plsc.parallel_loop(0, NW * 3, 1, unroll=4)(build)

    n_chunks = (NW * 3 * 2 * L) // _GCHUNK
    copies = [
        pltpu.async_copy(acc_hbm.at[gidx_v.at[pl.ds(j * _GCHUNK, _GCHUNK)]],
                         gath_v.at[pl.ds(j * _GCHUNK, _GCHUNK)], sem)
        for j in range(n_chunks)
    ]
    for c in copies:
        c.wait()

    zero = jnp.zeros((L,), jnp.float32)

    def red(k, carry):
        d0, d1, a0, a1, b0, b1 = carry
        base = k * 96
        d0 = d0 + gath_v[pl.ds(base, L)]
        d1 = d1 + gath_v[pl.ds(base + 16, L)]
        a0 = a0 + gath_v[pl.ds(base + 32, L)]
        a1 = a1 + gath_v[pl.ds(base + 48, L)]
        b0 = b0 + gath_v[pl.ds(base + 64, L)]
        b1 = b1 + gath_v[pl.ds(base + 80, L)]
        return d0, d1, a0, a1, b0, b1

    d0, d1, a0, a1, b0, b1 = lax.fori_loop(
        0, NW, red, (zero, zero, zero, zero, zero, zero))

    eps = jnp.float32(1e-16)
    bc0 = plsc.load_gather(bc_v, [jnp.zeros((L,), jnp.int32)])
    bc1 = plsc.load_gather(bc_v, [jnp.ones((L,), jnp.int32)])
    o00 = a0 / (d0 + eps) + bc0
    o01 = b0 / (d0 + eps) + bc1
    o10 = a1 / (d1 + eps) + bc0
    o11 = b1 / (d1 + eps) + bc1

    out_v[pl.ds(0, L)] = o00
    out_v[pl.ds(L, L)] = o10
    out_v[pl.ds(2 * L, L)] = o01
    out_v[pl.ds(3 * L, L)] = o11
    pltpu.sync_copy(out_v.at[pl.ds(0, BPW)], out_hbm.at[pl.ds(wid * BPW, BPW)])
    pltpu.sync_copy(out_v.at[pl.ds(BPW, BPW)],
                    out_hbm.at[pl.ds(B + wid * BPW, BPW)])


@jax.jit
def _select_pass(acc_flat, idx, bc):
    mesh = plsc.VectorSubcoreMesh(core_axis_name="c", subcore_axis_name="s",
                                  num_cores=NC, num_subcores=NS)
    return pl.kernel(
        _select_body,
        out_type=jax.ShapeDtypeStruct((2 * B,), jnp.float32),
        mesh=mesh,
        compiler_params=pltpu.CompilerParams(needs_layout_passes=False),
        scratch_types=[
            pltpu.VMEM((BPW,), jnp.int32),
            pltpu.VMEM((NW * 3 * 2 * L,), jnp.int32),
            pltpu.VMEM((NW * 3 * 2 * L,), jnp.float32),
            pltpu.VMEM((2 * BPW,), jnp.float32),
            pltpu.VMEM((2,), jnp.float32),
            pltpu.SemaphoreType.DMA,
        ],
    )(acc_flat, idx, bc)


def kernel(img1, img2, index, x, edge_index, W, a_src, a_dst, Wc, bc):
    f_flat = _fmat(x, W, a_src, a_dst, Wc.T.reshape(-1))
    acc = _edge_pass(f_flat, edge_index)
    idx = index.reshape(-1).astype(jnp.int32)
    out_flat = _select_pass(acc, idx, bc)
    return out_flat.reshape(2, B).T


# speedup vs baseline: 1.1571x; 1.0602x over previous
import functools

import jax
import jax.numpy as jnp
from jax import lax
from jax.experimental import pallas as pl
from jax.experimental.pallas import tpu as pltpu
from jax.experimental.pallas import tpu_sc as plsc

N_NODES = 10000
D_FEAT = 128
N_EDGES = 320000
B = 1024

NC = 2
NS = 16
L = 16
NW = NC * NS
EPW = N_EDGES // NW
BPW = B // NW


NPAD = 10240


def _fmat_body(x_ref, w_ref, as_ref, ad_ref, wcf_ref, f_ref):
    wcf = wcf_ref[...]
    ms = jnp.concatenate(
        [as_ref[...][:, None], ad_ref[...][:, None],
         wcf[0:256][:, None], wcf[256:512][:, None]], axis=1)
    m = jnp.dot(w_ref[...], ms, preferred_element_type=jnp.float32,
                precision=lax.Precision.HIGHEST)
    f = jnp.dot(x_ref[...], m, preferred_element_type=jnp.float32)
    ft = f.T
    for c in range(4):
        f_ref[pl.ds(c * NPAD, N_NODES)] = ft[c]


@jax.jit
def _fmat(x, w, a_src, a_dst, wc_flat):
    return pl.pallas_call(
        _fmat_body,
        out_shape=jax.ShapeDtypeStruct((4 * NPAD,), jnp.float32),
    )(x, w, a_src, a_dst, wc_flat)


ALEN = 10112


def _edge_body(f_hbm, ei_hbm, acc_hbm, f_v, ei_v, den_v, n0_v, n1_v, sem):
    wid = lax.axis_index("s") * NC + lax.axis_index("c")
    base = wid * EPW
    a0 = (base // 128) * 128
    off = base - a0

    cf = pltpu.async_copy(f_hbm, f_v, sem)
    ce = pltpu.async_copy(ei_hbm.at[:, pl.ds(a0, ALEN)], ei_v, sem)

    zero = jnp.zeros((L,), jnp.float32)

    def zbody(i):
        den_v[pl.ds(i * L, L)] = zero
        n0_v[pl.ds(i * L, L)] = zero
        n1_v[pl.ds(i * L, L)] = zero

    plsc.parallel_loop(0, N_NODES // L, 1, unroll=4)(zbody)
    cf.wait()
    ce.wait()

    def ebody(i):
        s = ei_v[0, pl.ds(off + i * L, L)]
        d = ei_v[1, pl.ds(off + i * L, L)]
        av = plsc.load_gather(f_v, [s])
        bv = plsc.load_gather(f_v, [d + NPAD])
        h0 = plsc.load_gather(f_v, [s + 2 * NPAD])
        h1 = plsc.load_gather(f_v, [s + 3 * NPAD])
        z = av + bv
        e = jnp.where(z >= 0.0, z, z * jnp.float32(0.2))
        ex = jnp.exp(e)
        plsc.addupdate_scatter(den_v, [d], ex)
        plsc.addupdate_scatter(n0_v, [d], ex * h0)
        plsc.addupdate_scatter(n1_v, [d], ex * h1)

    plsc.parallel_loop(0, EPW // L, 1, unroll=8)(ebody)

    row = wid * 3 * N_NODES
    pltpu.sync_copy(den_v, acc_hbm.at[pl.ds(row, N_NODES)])
    pltpu.sync_copy(n0_v, acc_hbm.at[pl.ds(row + N_NODES, N_NODES)])
    pltpu.sync_copy(n1_v, acc_hbm.at[pl.ds(row + 2 * N_NODES, N_NODES)])


@jax.jit
def _edge_pass(f_flat, edge_index):
    mesh = plsc.VectorSubcoreMesh(core_axis_name="c", subcore_axis_name="s",
                                  num_cores=NC, num_subcores=NS)
    return pl.kernel(
        _edge_body,
        out_type=jax.ShapeDtypeStruct((NW * 3 * N_NODES,), jnp.float32),
        mesh=mesh,
        compiler_params=pltpu.CompilerParams(needs_layout_passes=False),
        scratch_types=[
            pltpu.VMEM((4 * NPAD,), jnp.float32),
            pltpu.VMEM((2, ALEN), jnp.int32),
            pltpu.VMEM((N_NODES,), jnp.float32),
            pltpu.VMEM((N_NODES,), jnp.float32),
            pltpu.VMEM((N_NODES,), jnp.float32),
            pltpu.SemaphoreType.DMA,
        ],
    )(f_flat, edge_index)


_GCHUNK = 128


def _select_body(acc_hbm, idx_hbm, bc_hbm, out_hbm, idx_v, gidx_v, gath_v,
                 out_v, bc_v, sem):
    wid = lax.axis_index("s") * NC + lax.axis_index("c")
    pltpu.sync_copy(idx_hbm.at[pl.ds(wid * BPW, BPW)], idx_v)
    pltpu.sync_copy(bc_hbm, bc_v)

    i0 = idx_v[pl.ds(0, L)]
    i1 = idx_v[pl.ds(L, L)]

    def build(j):
        off = j * N_NODES
        gidx_v[pl.ds(j * 2 * L, L)] = i0 + off
        gidx_v[pl.ds(j * 2 * L + L, L)] = i1 + off

    plsc.parallel_loop(0, NW * 3, 1, unroll=4)(build)

    n_chunks = (NW * 3 * 2 * L) // _GCHUNK
    copies = [
        pltpu.async_copy(acc_hbm.at[gidx_v.at[pl.ds(j * _GCHUNK, _GCHUNK)]],
                         gath_v.at[pl.ds(j * _GCHUNK, _GCHUNK)], sem)
        for j in range(n_chunks)
    ]
    for c in copies:
        c.wait()

    zero = jnp.zeros((L,), jnp.float32)

    def red(k, carry):
        d0, d1, a0, a1, b0, b1 = carry
        base = k * 96
        d0 = d0 + gath_v[pl.ds(base, L)]
        d1 = d1 + gath_v[pl.ds(base + 16, L)]
        a0 = a0 + gath_v[pl.ds(base + 32, L)]
        a1 = a1 + gath_v[pl.ds(base + 48, L)]
        b0 = b0 + gath_v[pl.ds(base + 64, L)]
        b1 = b1 + gath_v[pl.ds(base + 80, L)]
        return d0, d1, a0, a1, b0, b1

    d0, d1, a0, a1, b0, b1 = lax.fori_loop(
        0, NW, red, (zero, zero, zero, zero, zero, zero))

    eps = jnp.float32(1e-16)
    bc0 = plsc.load_gather(bc_v, [jnp.zeros((L,), jnp.int32)])
    bc1 = plsc.load_gather(bc_v, [jnp.ones((L,), jnp.int32)])
    o00 = a0 / (d0 + eps) + bc0
    o01 = b0 / (d0 + eps) + bc1
    o10 = a1 / (d1 + eps) + bc0
    o11 = b1 / (d1 + eps) + bc1

    out_v[pl.ds(0, L)] = o00
    out_v[pl.ds(L, L)] = o10
    out_v[pl.ds(2 * L, L)] = o01
    out_v[pl.ds(3 * L, L)] = o11
    pltpu.sync_copy(out_v.at[pl.ds(0, BPW)], out_hbm.at[pl.ds(wid * BPW, BPW)])
    pltpu.sync_copy(out_v.at[pl.ds(BPW, BPW)],
                    out_hbm.at[pl.ds(B + wid * BPW, BPW)])


@jax.jit
def _select_pass(acc_flat, idx, bc):
    mesh = plsc.VectorSubcoreMesh(core_axis_name="c", subcore_axis_name="s",
                                  num_cores=NC, num_subcores=NS)
    return pl.kernel(
        _select_body,
        out_type=jax.ShapeDtypeStruct((2 * B,), jnp.float32),
        mesh=mesh,
        compiler_params=pltpu.CompilerParams(needs_layout_passes=False),
        scratch_types=[
            pltpu.VMEM((BPW,), jnp.int32),
            pltpu.VMEM((NW * 3 * 2 * L,), jnp.int32),
            pltpu.VMEM((NW * 3 * 2 * L,), jnp.float32),
            pltpu.VMEM((2 * BPW,), jnp.float32),
            pltpu.VMEM((2,), jnp.float32),
            pltpu.SemaphoreType.DMA,
        ],
    )(acc_flat, idx, bc)


def kernel(img1, img2, index, x, edge_index, W, a_src, a_dst, Wc, bc):
    f_flat = _fmat(x, W, a_src, a_dst, Wc.T.reshape(-1))
    acc = _edge_pass(f_flat, edge_index)
    idx = index.reshape(-1).astype(jnp.int32)
    out_flat = _select_pass(acc, idx, bc)
    return out_flat.reshape(2, B).T
